# Initial kernel scaffold; baseline (speedup 1.0000x reference)
import jax
import jax.numpy as jnp
from jax.experimental import pallas as pl


def kernel(positions, node_feats, senders, receivers, W, b):
    raise NotImplementedError("write your pallas kernel here")


# speedup vs baseline: 47.3484x; 47.3484x over previous
import functools

import jax
import jax.numpy as jnp
from jax import lax
from jax.experimental import pallas as pl
from jax.experimental.pallas import tpu as pltpu
from jax.experimental.pallas import tpu_sc as plsc

NC = 2
NS = 16
L = 16
CHUNK = 128

INV_SQRT3 = 0.57735026918962576
INV_SQRT2 = 0.70710678118654752


def _sc_aggregate(pos4, feats2, senders, receivers, n_nodes, n_edges):
    n_chunks = n_edges // CHUNK
    rows_per_tile = n_nodes // NS
    zrows = rows_per_tile // 10
    mesh = plsc.VectorSubcoreMesh(core_axis_name="c", subcore_axis_name="s",
                                  num_cores=NC, num_subcores=NS)

    @functools.partial(
        pl.kernel,
        out_type=jax.ShapeDtypeStruct((NC, n_nodes, 16), jnp.float32),
        mesh=mesh,
        scratch_types=[
            pltpu.VMEM((CHUNK,), jnp.int32),
            pltpu.VMEM((CHUNK,), jnp.int32),
            pltpu.VMEM((CHUNK,), jnp.int32),
            pltpu.VMEM((CHUNK,), jnp.int32),
            pltpu.VMEM((CHUNK,), jnp.int32),
            pltpu.VMEM((CHUNK, 16), jnp.float32),
            pltpu.VMEM((CHUNK, 16), jnp.float32),
            pltpu.VMEM((CHUNK, 16), jnp.float32),
            pltpu.VMEM((CHUNK, 16), jnp.float32),
            pltpu.VMEM((zrows, 16), jnp.float32),
            pltpu.VMEM_SHARED((n_nodes, 16), jnp.float32),
            pltpu.SemaphoreType.DMA,
        ],
        compiler_params=pltpu.CompilerParams(
            use_tc_tiling_on_sc=False, needs_layout_passes=False),
    )
    def body(pos_hbm, feats_hbm, send_hbm, recv_hbm, out_hbm,
             sidx, ridx, fidx, spv, rpv, pos_r, pos_s, feats, tp, stage,
             agg, sem):
        c = lax.axis_index("c")
        t = lax.axis_index("s")
        base_row = t * rows_per_tile

        zero16 = jnp.zeros((L,), jnp.float32)

        def zrow(i, _):
            stage[i] = zero16
            return 0
        lax.fori_loop(0, zrows, zrow, 0)

        def zcopy(j, _):
            pltpu.sync_copy(stage, agg.at[pl.ds(base_row + j * zrows, zrows)])
            return 0
        lax.fori_loop(0, rows_per_tile // zrows, zcopy, 0)
        plsc.subcore_barrier()

        def chunk_body(i, _):
            k = t + NS * i
            e0 = k * CHUNK
            pltpu.sync_copy(send_hbm.at[pl.ds(e0, CHUNK)], sidx)
            pltpu.sync_copy(recv_hbm.at[pl.ds(e0, CHUNK)], ridx)

            def fb(g, _):
                s16 = sidx[pl.ds(g * L, L)]
                r16 = ridx[pl.ds(g * L, L)]
                fidx[pl.ds(g * L, L)] = s16 * 2 + c
                spv[pl.ds(g * L, L)] = s16 >> 2
                rpv[pl.ds(g * L, L)] = r16 >> 2
                return 0
            lax.fori_loop(0, CHUNK // L, fb, 0)

            pltpu.async_copy(pos_hbm.at[rpv], pos_r, sem).wait()
            pltpu.async_copy(pos_hbm.at[spv], pos_s, sem).wait()
            pltpu.async_copy(feats_hbm.at[fidx], feats, sem).wait()

            def gb(g, _):
                rows = g * L + lax.iota(jnp.int32, L)
                par_r = (ridx[pl.ds(g * L, L)] & 3) * 4
                par_s = (sidx[pl.ds(g * L, L)] & 3) * 4
                dx = (plsc.load_gather(pos_r, [rows, par_r])
                      - plsc.load_gather(pos_s, [rows, par_s]))
                dy = (plsc.load_gather(pos_r, [rows, par_r + 1])
                      - plsc.load_gather(pos_s, [rows, par_s + 1]))
                dz = (plsc.load_gather(pos_r, [rows, par_r + 2])
                      - plsc.load_gather(pos_s, [rows, par_s + 2]))
                s2 = dx * dx + dy * dy + dz * dz
                bi = plsc.bitcast(s2, jnp.int32)
                bi = jnp.int32(0x5F3759DF) - (bi >> 1)
                y = plsc.bitcast(bi, jnp.float32)
                y = y * (1.5 - 0.5 * s2 * y * y)
                y = y * (1.5 - 0.5 * s2 * y * y)
                y = y * (1.5 - 0.5 * s2 * y * y)
                inv = 1.0 / (s2 * y + 1e-9)
                nx = dx * inv
                ny = dy * inv
                nz = dz * inv
                for f in range(4):
                    cw = [jnp.full((L,), 4 * f + q, jnp.int32) for q in range(4)]
                    x0 = plsc.load_gather(feats, [rows, cw[0]])
                    x1 = plsc.load_gather(feats, [rows, cw[1]])
                    x2 = plsc.load_gather(feats, [rows, cw[2]])
                    x3 = plsc.load_gather(feats, [rows, cw[3]])
                    dot = (x1 * nx + x2 * ny + x3 * nz) * INV_SQRT3
                    o0 = x0 + dot
                    o1 = x0 * nx + x1 + (x2 * nz - x3 * ny) * INV_SQRT2
                    o2 = x0 * ny + x2 + (x3 * nx - x1 * nz) * INV_SQRT2
                    o3 = x0 * nz + x3 + (x1 * ny - x2 * nx) * INV_SQRT2
                    plsc.store_scatter(tp, [rows, cw[0]], o0)
                    plsc.store_scatter(tp, [rows, cw[1]], o1)
                    plsc.store_scatter(tp, [rows, cw[2]], o2)
                    plsc.store_scatter(tp, [rows, cw[3]], o3)
                return 0
            lax.fori_loop(0, CHUNK // L, gb, 0)

            pltpu.sync_copy(tp, agg.at[ridx], add=True)
            return 0

        n_i = (n_chunks - t + NS - 1) // NS
        lax.fori_loop(0, n_i, chunk_body, 0)
        plsc.subcore_barrier()

        def flush(j, _):
            r0 = base_row + j * zrows
            pltpu.sync_copy(agg.at[pl.ds(r0, zrows)], stage)
            pltpu.sync_copy(stage, out_hbm.at[c, pl.ds(r0, zrows)])
            return 0
        lax.fori_loop(0, rows_per_tile // zrows, flush, 0)

    return body(pos4, feats2, senders, receivers)


def _tc_dense(agg2, bd, b32, n_nodes):
    rows = 2000
    grid = n_nodes // rows

    def body(a_ref, bd_ref, b_ref, o_ref):
        x = jnp.concatenate([a_ref[0], a_ref[1]], axis=-1)
        o_ref[...] = (jnp.dot(x, bd_ref[...],
                              preferred_element_type=jnp.float32)
                      + b_ref[...])

    return pl.pallas_call(
        body,
        grid=(grid,),
        in_specs=[
            pl.BlockSpec((NC, rows, 16), lambda i: (0, i, 0)),
            pl.BlockSpec((32, 32), lambda i: (0, 0)),
            pl.BlockSpec((1, 32), lambda i: (0, 0)),
        ],
        out_specs=pl.BlockSpec((rows, 32), lambda i: (i, 0)),
        out_shape=jax.ShapeDtypeStruct((n_nodes, 32), jnp.float32),
    )(agg2, bd, b32)


def kernel(positions, node_feats, senders, receivers, W, b):
    n_nodes = positions.shape[0]
    n_edges = senders.shape[0]
    n_f = node_feats.shape[1]
    pos4 = jnp.pad(positions, ((0, 0), (0, 1))).reshape(n_nodes // 4, 16)
    feats2 = node_feats.reshape(n_nodes * NC, n_f * 4 // NC)
    agg2 = _sc_aggregate(pos4, feats2, senders, receivers, n_nodes, n_edges)
    bd = jnp.kron(jnp.eye(n_f, dtype=W.dtype), W) * (1.0 / 16.0)
    b32 = jnp.tile(b, n_f).reshape(1, n_f * 4)
    out32 = _tc_dense(agg2, bd, b32, n_nodes)
    return out32.reshape(n_nodes, n_f, 4)


# speedup vs baseline: 71.4344x; 1.5087x over previous
import functools

import jax
import jax.numpy as jnp
from jax import lax
from jax.experimental import pallas as pl
from jax.experimental.pallas import tpu as pltpu
from jax.experimental.pallas import tpu_sc as plsc

NC = 2
NS = 16
L = 16
CHUNK = 128
NSLOT = 2

INV_SQRT3 = 0.57735026918962576
INV_SQRT2 = 0.70710678118654752


def _sc_aggregate(pos_flat, feats_flat, senders, receivers, n_nodes, n_edges):
    n_chunks = n_edges // CHUNK
    rows_per_tile = n_nodes // NS
    zrows = rows_per_tile // 10
    mesh = plsc.VectorSubcoreMesh(core_axis_name="c", subcore_axis_name="s",
                                  num_cores=NC, num_subcores=NS)

    idx_t = pltpu.VMEM((CHUNK,), jnp.int32)
    row_t = pltpu.VMEM((CHUNK, 16), jnp.float32)

    @functools.partial(
        pl.kernel,
        out_type=jax.ShapeDtypeStruct((NC, n_nodes, 16), jnp.float32),
        mesh=mesh,
        scratch_types=[
            [idx_t] * NSLOT,
            [idx_t] * NSLOT,
            [idx_t] * NSLOT,
            [idx_t] * NSLOT,
            [idx_t] * NSLOT,
            [row_t] * NSLOT,
            [row_t] * NSLOT,
            [row_t] * NSLOT,
            [row_t] * NSLOT,
            pltpu.VMEM((zrows, 16), jnp.float32),
            pltpu.VMEM_SHARED((n_nodes, 16), jnp.float32),
            [pltpu.SemaphoreType.DMA] * NSLOT,
            [pltpu.SemaphoreType.DMA] * NSLOT,
            [pltpu.SemaphoreType.DMA] * NSLOT,
        ],
        compiler_params=pltpu.CompilerParams(
            use_tc_tiling_on_sc=False, needs_layout_passes=False),
    )
    def body(pos_hbm, feats_hbm, send_hbm, recv_hbm, out_hbm,
             sidx, ridx, fidx, spv, rpv, pos_r, pos_s, feats, tp, stage,
             agg, sem_lin, sem_gat, sem_sca):
        c = lax.axis_index("c")
        t = lax.axis_index("s")
        base_row = t * rows_per_tile
        pos2d = pos_hbm
        feats2d = feats_hbm

        def chunk_of(i):
            return t + NS * i

        def issue_linear(i, s):
            e0 = chunk_of(i) * CHUNK
            pltpu.async_copy(send_hbm.at[pl.ds(e0, CHUNK)], sidx[s],
                             sem_lin[s])
            pltpu.async_copy(recv_hbm.at[pl.ds(e0, CHUNK)], ridx[s],
                             sem_lin[s])

        def wait_linear(i, s):
            e0 = chunk_of(i) * CHUNK
            pltpu.make_async_copy(send_hbm.at[pl.ds(e0, CHUNK)], sidx[s],
                                  sem_lin[s]).wait()
            pltpu.make_async_copy(recv_hbm.at[pl.ds(e0, CHUNK)], ridx[s],
                                  sem_lin[s]).wait()

        def derive(s):
            for g in range(CHUNK // L):
                s16 = sidx[s][pl.ds(g * L, L)]
                r16 = ridx[s][pl.ds(g * L, L)]
                fidx[s][pl.ds(g * L, L)] = s16 * 2 + c
                spv[s][pl.ds(g * L, L)] = s16 >> 2
                rpv[s][pl.ds(g * L, L)] = r16 >> 2

        def issue_gathers(s):
            pltpu.async_copy(pos2d.at[rpv[s]], pos_r[s], sem_gat[s])
            pltpu.async_copy(pos2d.at[spv[s]], pos_s[s], sem_gat[s])
            pltpu.async_copy(feats2d.at[fidx[s]], feats[s], sem_gat[s])

        def wait_gathers(s):
            pltpu.make_async_copy(pos2d.at[rpv[s]], pos_r[s],
                                  sem_gat[s]).wait()
            pltpu.make_async_copy(pos2d.at[spv[s]], pos_s[s],
                                  sem_gat[s]).wait()
            pltpu.make_async_copy(feats2d.at[fidx[s]], feats[s],
                                  sem_gat[s]).wait()

        def issue_scatter(s):
            pltpu.async_copy(tp[s], agg.at[ridx[s]], sem_sca[s], add=True)

        def wait_scatter(s):
            pltpu.make_async_copy(tp[s], agg.at[ridx[s]],
                                  sem_sca[s]).wait()

        def compute(s):
            for g in range(CHUNK // L):
                rows = g * L + lax.iota(jnp.int32, L)
                par_r = (ridx[s][pl.ds(g * L, L)] & 3) * 4
                par_s = (sidx[s][pl.ds(g * L, L)] & 3) * 4
                dx = (plsc.load_gather(pos_r[s], [rows, par_r])
                      - plsc.load_gather(pos_s[s], [rows, par_s]))
                dy = (plsc.load_gather(pos_r[s], [rows, par_r + 1])
                      - plsc.load_gather(pos_s[s], [rows, par_s + 1]))
                dz = (plsc.load_gather(pos_r[s], [rows, par_r + 2])
                      - plsc.load_gather(pos_s[s], [rows, par_s + 2]))
                s2 = dx * dx + dy * dy + dz * dz
                bi = plsc.bitcast(s2, jnp.int32)
                bi = jnp.int32(0x5F3759DF) - (bi >> 1)
                y = plsc.bitcast(bi, jnp.float32)
                y = y * (1.5 - 0.5 * s2 * y * y)
                y = y * (1.5 - 0.5 * s2 * y * y)
                y = y * (1.5 - 0.5 * s2 * y * y)
                inv = 1.0 / (s2 * y + 1e-9)
                nx = dx * inv
                ny = dy * inv
                nz = dz * inv
                for f in range(4):
                    cw = [jnp.full((L,), 4 * f + q, jnp.int32)
                          for q in range(4)]
                    x0 = plsc.load_gather(feats[s], [rows, cw[0]])
                    x1 = plsc.load_gather(feats[s], [rows, cw[1]])
                    x2 = plsc.load_gather(feats[s], [rows, cw[2]])
                    x3 = plsc.load_gather(feats[s], [rows, cw[3]])
                    dot = (x1 * nx + x2 * ny + x3 * nz) * INV_SQRT3
                    o0 = x0 + dot
                    o1 = x0 * nx + x1 + (x2 * nz - x3 * ny) * INV_SQRT2
                    o2 = x0 * ny + x2 + (x3 * nx - x1 * nz) * INV_SQRT2
                    o3 = x0 * nz + x3 + (x1 * ny - x2 * nx) * INV_SQRT2
                    plsc.store_scatter(tp[s], [rows, cw[0]], o0)
                    plsc.store_scatter(tp[s], [rows, cw[1]], o1)
                    plsc.store_scatter(tp[s], [rows, cw[2]], o2)
                    plsc.store_scatter(tp[s], [rows, cw[3]], o3)

        zero16 = jnp.zeros((L,), jnp.float32)

        def zrow(i, _):
            stage[i] = zero16
            return 0
        lax.fori_loop(0, zrows, zrow, 0)

        def zcopy(j, _):
            pltpu.sync_copy(stage, agg.at[pl.ds(base_row + j * zrows, zrows)])
            return 0
        lax.fori_loop(0, rows_per_tile // zrows, zcopy, 0)
        plsc.subcore_barrier()

        n_i = (n_chunks - t + NS - 1) // NS

        def pipe_body(i, j):
            s_a = j
            s_b = 1 - j

            @pl.when(jnp.logical_and(i + 1 >= 0, i + 1 < n_i))
            def _():
                @pl.when(i >= 1)
                def _():
                    wait_scatter(s_b)
                issue_linear(i + 1, s_b)
                wait_linear(i + 1, s_b)
                derive(s_b)
                issue_gathers(s_b)

            @pl.when(i >= 0)
            def _():
                wait_gathers(s_a)
                compute(s_a)
                issue_scatter(s_a)

        def pair(m, _):
            i0 = m * NSLOT - 1
            for jj in range(NSLOT):
                i = i0 + jj
                j = (jj + 1) % NSLOT
                @pl.when(i < n_i)
                def _():
                    pipe_body(i, j)
            return 0
        lax.fori_loop(0, (n_i + 1 + NSLOT - 1) // NSLOT, pair, 0)

        wait_scatter(0)
        wait_scatter(1)

        plsc.subcore_barrier()

        def flush(jj, _):
            r0 = base_row + jj * zrows
            pltpu.sync_copy(agg.at[pl.ds(r0, zrows)], stage)
            pltpu.sync_copy(stage, out_hbm.at[c, pl.ds(r0, zrows)])
            return 0
        lax.fori_loop(0, rows_per_tile // zrows, flush, 0)

    return body(pos_flat, feats_flat, senders, receivers)


def _tc_dense(agg2, bd, b32, n_nodes):
    rows = 2000
    grid = n_nodes // rows

    def body(a_ref, bd_ref, b_ref, o_ref):
        x = jnp.concatenate([a_ref[0], a_ref[1]], axis=-1)
        o_ref[...] = (jnp.dot(x, bd_ref[...],
                              preferred_element_type=jnp.float32)
                      + b_ref[...])

    return pl.pallas_call(
        body,
        grid=(grid,),
        in_specs=[
            pl.BlockSpec((NC, rows, 16), lambda i: (0, i, 0)),
            pl.BlockSpec((32, 32), lambda i: (0, 0)),
            pl.BlockSpec((1, 32), lambda i: (0, 0)),
        ],
        out_specs=pl.BlockSpec((rows, 32), lambda i: (i, 0)),
        out_shape=jax.ShapeDtypeStruct((n_nodes, 32), jnp.float32),
    )(agg2, bd, b32)


def kernel(positions, node_feats, senders, receivers, W, b):
    n_nodes = positions.shape[0]
    n_edges = senders.shape[0]
    n_f = node_feats.shape[1]
    pos4 = jnp.pad(positions, ((0, 0), (0, 1))).reshape(n_nodes // 4, 16)
    feats2 = node_feats.reshape(n_nodes * NC, n_f * 4 // NC)
    agg2 = _sc_aggregate(pos4, feats2, senders, receivers,
                         n_nodes, n_edges)
    bd = jnp.kron(jnp.eye(n_f, dtype=W.dtype), W) * (1.0 / 16.0)
    b32 = jnp.tile(b, n_f).reshape(1, n_f * 4)
    out32 = _tc_dense(agg2, bd, b32, n_nodes)
    return out32.reshape(n_nodes, n_f, 4)


# speedup vs baseline: 83.8202x; 1.1734x over previous
import functools

import jax
import jax.numpy as jnp
from jax import lax
from jax.experimental import pallas as pl
from jax.experimental.pallas import tpu as pltpu
from jax.experimental.pallas import tpu_sc as plsc

NC = 2
NS = 16
L = 16
CHUNK = 128
NSLOT = 2

INV_SQRT3 = 0.57735026918962576
INV_SQRT2 = 0.70710678118654752


def _sc_aggregate(pos_flat, feats_flat, senders, receivers, n_nodes, n_edges):
    n_chunks = n_edges // CHUNK
    rows_per_tile = n_nodes // NS
    zrows = rows_per_tile // 10
    mesh = plsc.VectorSubcoreMesh(core_axis_name="c", subcore_axis_name="s",
                                  num_cores=NC, num_subcores=NS)

    idx_t = pltpu.VMEM((CHUNK,), jnp.int32)
    row_t = pltpu.VMEM((CHUNK, 16), jnp.float32)

    @functools.partial(
        pl.kernel,
        out_type=jax.ShapeDtypeStruct((NC, n_nodes, 16), jnp.float32),
        mesh=mesh,
        scratch_types=[
            [idx_t] * NSLOT,
            [idx_t] * NSLOT,
            [idx_t] * NSLOT,
            [idx_t] * NSLOT,
            [idx_t] * NSLOT,
            [idx_t] * NSLOT,
            [idx_t] * NSLOT,
            [idx_t] * NSLOT,
            [row_t] * NSLOT,
            [row_t] * NSLOT,
            [row_t] * NSLOT,
            [row_t] * NSLOT,
            pltpu.VMEM((zrows, 16), jnp.float32),
            pltpu.VMEM_SHARED((n_nodes, 16), jnp.float32),
            [pltpu.SemaphoreType.DMA] * NSLOT,
            [pltpu.SemaphoreType.DMA] * NSLOT,
            [pltpu.SemaphoreType.DMA] * NSLOT,
        ],
        compiler_params=pltpu.CompilerParams(
            use_tc_tiling_on_sc=False, needs_layout_passes=False),
    )
    def body(pos_hbm, feats_hbm, send_hbm, recv_hbm, out_hbm,
             sidx, ridx, fidx, spv, rpv, parbr, parbs, rsca,
             pos_r, pos_s, feats, tp, stage,
             agg, sem_lin, sem_gat, sem_sca):
        c = lax.axis_index("c")
        t = lax.axis_index("s")
        base_row = t * rows_per_tile
        pos2d = pos_hbm
        feats2d = feats_hbm

        def chunk_of(i):
            return t + NS * i

        def issue_linear(i, s):
            e0 = chunk_of(i) * CHUNK
            pltpu.async_copy(send_hbm.at[pl.ds(e0, CHUNK)], sidx[s],
                             sem_lin[s])
            pltpu.async_copy(recv_hbm.at[pl.ds(e0, CHUNK)], ridx[s],
                             sem_lin[s])

        def wait_linear(i, s):
            e0 = chunk_of(i) * CHUNK
            pltpu.make_async_copy(send_hbm.at[pl.ds(e0, CHUNK)], sidx[s],
                                  sem_lin[s]).wait()
            pltpu.make_async_copy(recv_hbm.at[pl.ds(e0, CHUNK)], ridx[s],
                                  sem_lin[s]).wait()

        def derive(s):
            for g in range(CHUNK // L):
                sl = pl.ds(g * L, L)
                s16 = sidx[s][sl]
                r16 = ridx[s][sl]
                fidx[s][sl] = s16 * 2 + c
                spv[s][sl] = s16 >> 2
                rpv[s][sl] = r16 >> 2
                parbs[s][sl] = (s16 & 3) * 4
                parbr[s][sl] = (r16 & 3) * 4
                rsca[s][sl] = r16

        def issue_gathers(s):
            pltpu.async_copy(pos2d.at[rpv[s]], pos_r[s], sem_gat[s])
            pltpu.async_copy(pos2d.at[spv[s]], pos_s[s], sem_gat[s])
            pltpu.async_copy(feats2d.at[fidx[s]], feats[s], sem_gat[s])

        def wait_gathers(s):
            pltpu.make_async_copy(pos2d.at[rpv[s]], pos_r[s],
                                  sem_gat[s]).wait()
            pltpu.make_async_copy(pos2d.at[spv[s]], pos_s[s],
                                  sem_gat[s]).wait()
            pltpu.make_async_copy(feats2d.at[fidx[s]], feats[s],
                                  sem_gat[s]).wait()

        def issue_scatter(s):
            pltpu.async_copy(tp[s], agg.at[rsca[s]], sem_sca[s], add=True)

        def wait_scatter(s):
            pltpu.make_async_copy(tp[s], agg.at[rsca[s]],
                                  sem_sca[s]).wait()

        def compute(s):
            for g in range(CHUNK // L):
                rows = g * L + lax.iota(jnp.int32, L)
                par_r = parbr[s][pl.ds(g * L, L)]
                par_s = parbs[s][pl.ds(g * L, L)]
                dx = (plsc.load_gather(pos_r[s], [rows, par_r])
                      - plsc.load_gather(pos_s[s], [rows, par_s]))
                dy = (plsc.load_gather(pos_r[s], [rows, par_r + 1])
                      - plsc.load_gather(pos_s[s], [rows, par_s + 1]))
                dz = (plsc.load_gather(pos_r[s], [rows, par_r + 2])
                      - plsc.load_gather(pos_s[s], [rows, par_s + 2]))
                s2 = dx * dx + dy * dy + dz * dz
                bi = plsc.bitcast(s2, jnp.int32)
                bi = jnp.int32(0x5F3759DF) - (bi >> 1)
                y = plsc.bitcast(bi, jnp.float32)
                y = y * (1.5 - 0.5 * s2 * y * y)
                y = y * (1.5 - 0.5 * s2 * y * y)
                y = y * (1.5 - 0.5 * s2 * y * y)
                inv = 1.0 / (s2 * y + 1e-9)
                nx = dx * inv
                ny = dy * inv
                nz = dz * inv
                for f in range(4):
                    cw = [jnp.full((L,), 4 * f + q, jnp.int32)
                          for q in range(4)]
                    x0 = plsc.load_gather(feats[s], [rows, cw[0]])
                    x1 = plsc.load_gather(feats[s], [rows, cw[1]])
                    x2 = plsc.load_gather(feats[s], [rows, cw[2]])
                    x3 = plsc.load_gather(feats[s], [rows, cw[3]])
                    dot = (x1 * nx + x2 * ny + x3 * nz) * INV_SQRT3
                    o0 = x0 + dot
                    o1 = x0 * nx + x1 + (x2 * nz - x3 * ny) * INV_SQRT2
                    o2 = x0 * ny + x2 + (x3 * nx - x1 * nz) * INV_SQRT2
                    o3 = x0 * nz + x3 + (x1 * ny - x2 * nx) * INV_SQRT2
                    plsc.store_scatter(tp[s], [rows, cw[0]], o0)
                    plsc.store_scatter(tp[s], [rows, cw[1]], o1)
                    plsc.store_scatter(tp[s], [rows, cw[2]], o2)
                    plsc.store_scatter(tp[s], [rows, cw[3]], o3)

        zero16 = jnp.zeros((L,), jnp.float32)

        def zrow(i, _):
            stage[i] = zero16
            return 0
        lax.fori_loop(0, zrows, zrow, 0)

        def zcopy(j, _):
            pltpu.sync_copy(stage, agg.at[pl.ds(base_row + j * zrows, zrows)])
            return 0
        lax.fori_loop(0, rows_per_tile // zrows, zcopy, 0)
        plsc.subcore_barrier()

        n_i = (n_chunks - t + NS - 1) // NS

        def pipe_body(i, j):
            s_a = j
            s_b = 1 - j

            @pl.when(jnp.logical_and(i + 1 >= 0, i + 1 < n_i))
            def _():
                @pl.when(i >= 1)
                def _():
                    wait_scatter(s_b)
                @pl.when(i == -1)
                def _():
                    issue_linear(0, s_b)
                wait_linear(i + 1, s_b)
                derive(s_b)
                issue_gathers(s_b)
                @pl.when(i + 2 < n_i)
                def _():
                    issue_linear(i + 2, s_a)

            @pl.when(i >= 0)
            def _():
                wait_gathers(s_a)
                compute(s_a)
                issue_scatter(s_a)

        def pair(m, _):
            i0 = m * NSLOT - 1
            for jj in range(NSLOT):
                i = i0 + jj
                j = (jj + 1) % NSLOT
                @pl.when(i < n_i)
                def _():
                    pipe_body(i, j)
            return 0
        lax.fori_loop(0, (n_i + 1 + NSLOT - 1) // NSLOT, pair, 0)

        wait_scatter(0)
        wait_scatter(1)

        plsc.subcore_barrier()

        def flush(jj, _):
            r0 = base_row + jj * zrows
            pltpu.sync_copy(agg.at[pl.ds(r0, zrows)], stage)
            pltpu.sync_copy(stage, out_hbm.at[c, pl.ds(r0, zrows)])
            return 0
        lax.fori_loop(0, rows_per_tile // zrows, flush, 0)

    return body(pos_flat, feats_flat, senders, receivers)


def _tc_dense(agg2, bd, b32, n_nodes):
    rows = 2000
    grid = n_nodes // rows

    def body(a_ref, bd_ref, b_ref, o_ref):
        x = jnp.concatenate([a_ref[0], a_ref[1]], axis=-1)
        o_ref[...] = (jnp.dot(x, bd_ref[...],
                              preferred_element_type=jnp.float32)
                      + b_ref[...])

    return pl.pallas_call(
        body,
        grid=(grid,),
        in_specs=[
            pl.BlockSpec((NC, rows, 16), lambda i: (0, i, 0)),
            pl.BlockSpec((32, 32), lambda i: (0, 0)),
            pl.BlockSpec((1, 32), lambda i: (0, 0)),
        ],
        out_specs=pl.BlockSpec((rows, 32), lambda i: (i, 0)),
        out_shape=jax.ShapeDtypeStruct((n_nodes, 32), jnp.float32),
    )(agg2, bd, b32)


def kernel(positions, node_feats, senders, receivers, W, b):
    n_nodes = positions.shape[0]
    n_edges = senders.shape[0]
    n_f = node_feats.shape[1]
    pos4 = jnp.pad(positions, ((0, 0), (0, 1))).reshape(n_nodes // 4, 16)
    feats2 = node_feats.reshape(n_nodes * NC, n_f * 4 // NC)
    agg2 = _sc_aggregate(pos4, feats2, senders, receivers,
                         n_nodes, n_edges)
    bd = jnp.kron(jnp.eye(n_f, dtype=W.dtype), W) * (1.0 / 16.0)
    b32 = jnp.tile(b, n_f).reshape(1, n_f * 4)
    out32 = _tc_dense(agg2, bd, b32, n_nodes)
    return out32.reshape(n_nodes, n_f, 4)


# speedup vs baseline: 86.7532x; 1.0350x over previous
import functools

import jax
import jax.numpy as jnp
from jax import lax
from jax.experimental import pallas as pl
from jax.experimental.pallas import tpu as pltpu
from jax.experimental.pallas import tpu_sc as plsc

NC = 2
NS = 16
L = 16
CHUNK = 128
NSLOT = 2

INV_SQRT3 = 0.57735026918962576
INV_SQRT2 = 0.70710678118654752


def _sc_aggregate(pos_flat, feats_flat, senders, receivers, n_nodes, n_edges):
    n_chunks = n_edges // CHUNK
    rows_per_tile = n_nodes // NS
    zrows = rows_per_tile // 10
    mesh = plsc.VectorSubcoreMesh(core_axis_name="c", subcore_axis_name="s",
                                  num_cores=NC, num_subcores=NS)

    idx_t = pltpu.VMEM((CHUNK,), jnp.int32)
    row_t = pltpu.VMEM((CHUNK, 16), jnp.float32)

    @functools.partial(
        pl.kernel,
        out_type=jax.ShapeDtypeStruct((NC, n_nodes, 16), jnp.float32),
        mesh=mesh,
        scratch_types=[
            [idx_t] * NSLOT,
            [idx_t] * NSLOT,
            [idx_t] * NSLOT,
            [idx_t] * NSLOT,
            [idx_t] * NSLOT,
            [idx_t] * NSLOT,
            [idx_t] * NSLOT,
            [idx_t] * NSLOT,
            [row_t] * NSLOT,
            [row_t] * NSLOT,
            [row_t] * NSLOT,
            [row_t] * NSLOT,
            pltpu.VMEM((zrows, 16), jnp.float32),
            pltpu.VMEM_SHARED((n_nodes, 16), jnp.float32),
            [pltpu.SemaphoreType.DMA] * NSLOT,
            [pltpu.SemaphoreType.DMA] * NSLOT,
            [pltpu.SemaphoreType.DMA] * NSLOT,
        ],
        compiler_params=pltpu.CompilerParams(
            use_tc_tiling_on_sc=False, needs_layout_passes=False),
    )
    def body(pos_hbm, feats_hbm, send_hbm, recv_hbm, out_hbm,
             sidx, ridx, fidx, spv, rpv, parbr, parbs, rsca,
             pos_r, pos_s, feats, tp, stage,
             agg, sem_lin, sem_gat, sem_sca):
        c = lax.axis_index("c")
        t = lax.axis_index("s")
        base_row = t * rows_per_tile
        pos2d = pos_hbm
        feats2d = feats_hbm

        def chunk_of(i):
            return t + NS * i

        def issue_linear(i, s):
            e0 = chunk_of(i) * CHUNK
            pltpu.async_copy(send_hbm.at[pl.ds(e0, CHUNK)], sidx[s],
                             sem_lin[s])
            pltpu.async_copy(recv_hbm.at[pl.ds(e0, CHUNK)], ridx[s],
                             sem_lin[s])

        def wait_linear(i, s):
            e0 = chunk_of(i) * CHUNK
            pltpu.make_async_copy(send_hbm.at[pl.ds(e0, CHUNK)], sidx[s],
                                  sem_lin[s]).wait()
            pltpu.make_async_copy(recv_hbm.at[pl.ds(e0, CHUNK)], ridx[s],
                                  sem_lin[s]).wait()

        def derive_gidx(s):
            for g in range(CHUNK // L):
                sl = pl.ds(g * L, L)
                s16 = sidx[s][sl]
                r16 = ridx[s][sl]
                fidx[s][sl] = s16 * 2 + c
                spv[s][sl] = s16 >> 2
                rpv[s][sl] = r16 >> 2

        def derive_sca(s):
            for g in range(CHUNK // L):
                sl = pl.ds(g * L, L)
                s16 = sidx[s][sl]
                r16 = ridx[s][sl]
                parbs[s][sl] = (s16 & 3) * 4
                parbr[s][sl] = (r16 & 3) * 4
                rsca[s][sl] = r16

        def issue_gathers(s):
            pltpu.async_copy(pos2d.at[rpv[s]], pos_r[s], sem_gat[s])
            pltpu.async_copy(pos2d.at[spv[s]], pos_s[s], sem_gat[s])
            pltpu.async_copy(feats2d.at[fidx[s]], feats[s], sem_gat[s])

        def wait_gathers(s):
            pltpu.make_async_copy(pos2d.at[rpv[s]], pos_r[s],
                                  sem_gat[s]).wait()
            pltpu.make_async_copy(pos2d.at[spv[s]], pos_s[s],
                                  sem_gat[s]).wait()
            pltpu.make_async_copy(feats2d.at[fidx[s]], feats[s],
                                  sem_gat[s]).wait()

        def issue_scatter(s):
            pltpu.async_copy(tp[s], agg.at[rsca[s]], sem_sca[s], add=True)

        def wait_scatter(s):
            pltpu.make_async_copy(tp[s], agg.at[rsca[s]],
                                  sem_sca[s]).wait()

        def compute(s):
            for g in range(CHUNK // L):
                rows = g * L + lax.iota(jnp.int32, L)
                par_r = parbr[s][pl.ds(g * L, L)]
                par_s = parbs[s][pl.ds(g * L, L)]
                dx = (plsc.load_gather(pos_r[s], [rows, par_r])
                      - plsc.load_gather(pos_s[s], [rows, par_s]))
                dy = (plsc.load_gather(pos_r[s], [rows, par_r + 1])
                      - plsc.load_gather(pos_s[s], [rows, par_s + 1]))
                dz = (plsc.load_gather(pos_r[s], [rows, par_r + 2])
                      - plsc.load_gather(pos_s[s], [rows, par_s + 2]))
                s2 = dx * dx + dy * dy + dz * dz
                bi = plsc.bitcast(s2, jnp.int32)
                bi = jnp.int32(0x5F3759DF) - (bi >> 1)
                y = plsc.bitcast(bi, jnp.float32)
                y = y * (1.5 - 0.5 * s2 * y * y)
                y = y * (1.5 - 0.5 * s2 * y * y)
                y = y * (1.5 - 0.5 * s2 * y * y)
                inv = 1.0 / (s2 * y + 1e-9)
                nx = dx * inv
                ny = dy * inv
                nz = dz * inv
                for f in range(4):
                    cw = [jnp.full((L,), 4 * f + q, jnp.int32)
                          for q in range(4)]
                    x0 = plsc.load_gather(feats[s], [rows, cw[0]])
                    x1 = plsc.load_gather(feats[s], [rows, cw[1]])
                    x2 = plsc.load_gather(feats[s], [rows, cw[2]])
                    x3 = plsc.load_gather(feats[s], [rows, cw[3]])
                    dot = (x1 * nx + x2 * ny + x3 * nz) * INV_SQRT3
                    o0 = x0 + dot
                    o1 = x0 * nx + x1 + (x2 * nz - x3 * ny) * INV_SQRT2
                    o2 = x0 * ny + x2 + (x3 * nx - x1 * nz) * INV_SQRT2
                    o3 = x0 * nz + x3 + (x1 * ny - x2 * nx) * INV_SQRT2
                    plsc.store_scatter(tp[s], [rows, cw[0]], o0)
                    plsc.store_scatter(tp[s], [rows, cw[1]], o1)
                    plsc.store_scatter(tp[s], [rows, cw[2]], o2)
                    plsc.store_scatter(tp[s], [rows, cw[3]], o3)

        zero16 = jnp.zeros((L,), jnp.float32)

        def zrow(i, _):
            stage[i] = zero16
            return 0
        lax.fori_loop(0, zrows, zrow, 0)

        def zcopy(j, _):
            pltpu.sync_copy(stage, agg.at[pl.ds(base_row + j * zrows, zrows)])
            return 0
        lax.fori_loop(0, rows_per_tile // zrows, zcopy, 0)
        plsc.subcore_barrier()

        n_i = (n_chunks - t + NS - 1) // NS

        def pipe_body(i, j):
            s_a = j
            s_b = 1 - j

            @pl.when(jnp.logical_and(i + 1 >= 0, i + 1 < n_i))
            def _():
                @pl.when(i <= 0)
                def _():
                    issue_linear(i + 1, s_b)
                wait_linear(i + 1, s_b)
                derive_gidx(s_b)
                issue_gathers(s_b)

            @pl.when(i >= 0)
            def _():
                @pl.when(i >= 2)
                def _():
                    wait_scatter(s_a)
                derive_sca(s_a)
                @pl.when(i + 2 < n_i)
                def _():
                    issue_linear(i + 2, s_a)
                wait_gathers(s_a)
                compute(s_a)
                issue_scatter(s_a)

        def pair(m, _):
            i0 = m * NSLOT - 1
            for jj in range(NSLOT):
                i = i0 + jj
                j = (jj + 1) % NSLOT
                @pl.when(i < n_i)
                def _():
                    pipe_body(i, j)
            return 0
        lax.fori_loop(0, (n_i + 1 + NSLOT - 1) // NSLOT, pair, 0)

        wait_scatter(0)
        wait_scatter(1)

        plsc.subcore_barrier()

        def flush(jj, _):
            r0 = base_row + jj * zrows
            pltpu.sync_copy(agg.at[pl.ds(r0, zrows)], stage)
            pltpu.sync_copy(stage, out_hbm.at[c, pl.ds(r0, zrows)])
            return 0
        lax.fori_loop(0, rows_per_tile // zrows, flush, 0)

    return body(pos_flat, feats_flat, senders, receivers)


def _tc_dense(agg2, bd, b32, n_nodes):
    rows = 2000
    grid = n_nodes // rows

    def body(a_ref, bd_ref, b_ref, o_ref):
        x = jnp.concatenate([a_ref[0], a_ref[1]], axis=-1)
        o_ref[...] = (jnp.dot(x, bd_ref[...],
                              preferred_element_type=jnp.float32)
                      + b_ref[...])

    return pl.pallas_call(
        body,
        grid=(grid,),
        in_specs=[
            pl.BlockSpec((NC, rows, 16), lambda i: (0, i, 0)),
            pl.BlockSpec((32, 32), lambda i: (0, 0)),
            pl.BlockSpec((1, 32), lambda i: (0, 0)),
        ],
        out_specs=pl.BlockSpec((rows, 32), lambda i: (i, 0)),
        out_shape=jax.ShapeDtypeStruct((n_nodes, 32), jnp.float32),
    )(agg2, bd, b32)


def kernel(positions, node_feats, senders, receivers, W, b):
    n_nodes = positions.shape[0]
    n_edges = senders.shape[0]
    n_f = node_feats.shape[1]
    pos4 = jnp.pad(positions, ((0, 0), (0, 1))).reshape(n_nodes // 4, 16)
    feats2 = node_feats.reshape(n_nodes * NC, n_f * 4 // NC)
    agg2 = _sc_aggregate(pos4, feats2, senders, receivers,
                         n_nodes, n_edges)
    bd = jnp.kron(jnp.eye(n_f, dtype=W.dtype), W) * (1.0 / 16.0)
    b32 = jnp.tile(b, n_f).reshape(1, n_f * 4)
    out32 = _tc_dense(agg2, bd, b32, n_nodes)
    return out32.reshape(n_nodes, n_f, 4)


# speedup vs baseline: 102.1756x; 1.1778x over previous
import functools

import jax
import jax.numpy as jnp
from jax import lax
from jax.experimental import pallas as pl
from jax.experimental.pallas import tpu as pltpu
from jax.experimental.pallas import tpu_sc as plsc

NC = 2
NS = 16
L = 16
CHUNK = 128
NSLOT = 2

INV_SQRT3 = 0.57735026918962576
INV_SQRT2 = 0.70710678118654752


def _sc_aggregate(pos_flat, feats_flat, senders, receivers, n_nodes, n_edges):
    n_chunks = n_edges // CHUNK
    rows_per_tile = n_nodes // NS
    zrows = rows_per_tile // 10
    mesh = plsc.VectorSubcoreMesh(core_axis_name="c", subcore_axis_name="s",
                                  num_cores=NC, num_subcores=NS)

    idx_t = pltpu.VMEM((CHUNK,), jnp.int32)
    row_t = pltpu.VMEM((CHUNK, 16), jnp.float32)

    @functools.partial(
        pl.kernel,
        out_type=jax.ShapeDtypeStruct((NC, n_nodes, 16), jnp.float32),
        mesh=mesh,
        scratch_types=[
            [idx_t] * NSLOT,
            [idx_t] * NSLOT,
            [idx_t] * NSLOT,
            [idx_t] * NSLOT,
            [idx_t] * NSLOT,
            [idx_t] * NSLOT,
            [idx_t] * NSLOT,
            [idx_t] * NSLOT,
            [row_t] * NSLOT,
            [row_t] * NSLOT,
            [row_t] * NSLOT,
            [row_t] * NSLOT,
            pltpu.VMEM((zrows, 16), jnp.float32),
            pltpu.VMEM_SHARED((n_nodes, 16), jnp.float32),
            [pltpu.SemaphoreType.DMA] * NSLOT,
            [pltpu.SemaphoreType.DMA] * NSLOT,
            [pltpu.SemaphoreType.DMA] * NSLOT,
        ],
        compiler_params=pltpu.CompilerParams(
            use_tc_tiling_on_sc=False, needs_layout_passes=False),
    )
    def body(pos_hbm, feats_hbm, send_hbm, recv_hbm, out_hbm,
             sidx, ridx, fidx, spv, rpv, parbr, parbs, rsca,
             pos_r, pos_s, feats, tp, stage,
             agg, sem_lin, sem_gat, sem_sca):
        c = lax.axis_index("c")
        t = lax.axis_index("s")
        base_row = t * rows_per_tile
        pos2d = pos_hbm
        feats2d = feats_hbm

        def chunk_of(i):
            return t + NS * i

        def issue_linear(i, s):
            e0 = chunk_of(i) * CHUNK
            pltpu.async_copy(send_hbm.at[pl.ds(e0, CHUNK)], sidx[s],
                             sem_lin[s])
            pltpu.async_copy(recv_hbm.at[pl.ds(e0, CHUNK)], ridx[s],
                             sem_lin[s])

        def wait_linear(i, s):
            e0 = chunk_of(i) * CHUNK
            pltpu.make_async_copy(send_hbm.at[pl.ds(e0, CHUNK)], sidx[s],
                                  sem_lin[s]).wait()
            pltpu.make_async_copy(recv_hbm.at[pl.ds(e0, CHUNK)], ridx[s],
                                  sem_lin[s]).wait()

        def derive_gidx(s):
            def gbody(g, _):
                sl = pl.ds(g * L, L)
                s16 = sidx[s][sl]
                r16 = ridx[s][sl]
                fidx[s][sl] = s16 * 2 + c
                spv[s][sl] = s16 >> 2
                rpv[s][sl] = r16 >> 2
                return 0
            lax.fori_loop(0, CHUNK // L, gbody, 0)

        def derive_sca(s):
            def gbody(g, _):
                sl = pl.ds(g * L, L)
                s16 = sidx[s][sl]
                r16 = ridx[s][sl]
                parbs[s][sl] = (s16 & 3) * 4
                parbr[s][sl] = (r16 & 3) * 4
                rsca[s][sl] = r16
                return 0
            lax.fori_loop(0, CHUNK // L, gbody, 0)

        def issue_gathers(s):
            pltpu.async_copy(pos2d.at[rpv[s]], pos_r[s], sem_gat[s])
            pltpu.async_copy(pos2d.at[spv[s]], pos_s[s], sem_gat[s])
            pltpu.async_copy(feats2d.at[fidx[s]], feats[s], sem_gat[s])

        def wait_gathers(s):
            pltpu.make_async_copy(pos2d.at[rpv[s]], pos_r[s],
                                  sem_gat[s]).wait()
            pltpu.make_async_copy(pos2d.at[spv[s]], pos_s[s],
                                  sem_gat[s]).wait()
            pltpu.make_async_copy(feats2d.at[fidx[s]], feats[s],
                                  sem_gat[s]).wait()

        def issue_scatter(s):
            pltpu.async_copy(tp[s], agg.at[rsca[s]], sem_sca[s], add=True)

        def wait_scatter(s):
            pltpu.make_async_copy(tp[s], agg.at[rsca[s]],
                                  sem_sca[s]).wait()

        def compute(s):
            def gbody(g, _):
                rows = g * L + lax.iota(jnp.int32, L)
                par_r = parbr[s][pl.ds(g * L, L)]
                par_s = parbs[s][pl.ds(g * L, L)]
                dx = (plsc.load_gather(pos_r[s], [rows, par_r])
                      - plsc.load_gather(pos_s[s], [rows, par_s]))
                dy = (plsc.load_gather(pos_r[s], [rows, par_r + 1])
                      - plsc.load_gather(pos_s[s], [rows, par_s + 1]))
                dz = (plsc.load_gather(pos_r[s], [rows, par_r + 2])
                      - plsc.load_gather(pos_s[s], [rows, par_s + 2]))
                s2 = dx * dx + dy * dy + dz * dz
                bi = plsc.bitcast(s2, jnp.int32)
                bi = jnp.int32(0x5F3759DF) - (bi >> 1)
                y = plsc.bitcast(bi, jnp.float32)
                y = y * (1.5 - 0.5 * s2 * y * y)
                y = y * (1.5 - 0.5 * s2 * y * y)
                y = y * (1.5 - 0.5 * s2 * y * y)
                inv = 1.0 / (s2 * y + 1e-9)
                nx = dx * inv
                ny = dy * inv
                nz = dz * inv
                for f in range(4):
                    cw = [jnp.full((L,), 4 * f + q, jnp.int32)
                          for q in range(4)]
                    x0 = plsc.load_gather(feats[s], [rows, cw[0]])
                    x1 = plsc.load_gather(feats[s], [rows, cw[1]])
                    x2 = plsc.load_gather(feats[s], [rows, cw[2]])
                    x3 = plsc.load_gather(feats[s], [rows, cw[3]])
                    dot = (x1 * nx + x2 * ny + x3 * nz) * INV_SQRT3
                    o0 = x0 + dot
                    o1 = x0 * nx + x1 + (x2 * nz - x3 * ny) * INV_SQRT2
                    o2 = x0 * ny + x2 + (x3 * nx - x1 * nz) * INV_SQRT2
                    o3 = x0 * nz + x3 + (x1 * ny - x2 * nx) * INV_SQRT2
                    plsc.store_scatter(tp[s], [rows, cw[0]], o0)
                    plsc.store_scatter(tp[s], [rows, cw[1]], o1)
                    plsc.store_scatter(tp[s], [rows, cw[2]], o2)
                    plsc.store_scatter(tp[s], [rows, cw[3]], o3)
                return 0
            lax.fori_loop(0, CHUNK // L, gbody, 0)

        zero16 = jnp.zeros((L,), jnp.float32)

        def zrow(i, _):
            stage[i] = zero16
            return 0
        lax.fori_loop(0, zrows, zrow, 0)

        def zcopy(j, _):
            pltpu.sync_copy(stage, agg.at[pl.ds(base_row + j * zrows, zrows)])
            return 0
        lax.fori_loop(0, rows_per_tile // zrows, zcopy, 0)
        plsc.subcore_barrier()

        n_i = (n_chunks - t + NS - 1) // NS

        def pipe_body(i, j):
            s_a = j
            s_b = 1 - j

            @pl.when(jnp.logical_and(i + 1 >= 0, i + 1 < n_i))
            def _():
                @pl.when(i <= 0)
                def _():
                    issue_linear(i + 1, s_b)
                wait_linear(i + 1, s_b)
                derive_gidx(s_b)
                issue_gathers(s_b)

            @pl.when(i >= 0)
            def _():
                @pl.when(i >= 2)
                def _():
                    wait_scatter(s_a)
                derive_sca(s_a)
                @pl.when(i + 2 < n_i)
                def _():
                    issue_linear(i + 2, s_a)
                wait_gathers(s_a)
                compute(s_a)
                issue_scatter(s_a)

        def pair(m, _):
            i0 = m * NSLOT - 1
            for jj in range(NSLOT):
                i = i0 + jj
                j = (jj + 1) % NSLOT
                @pl.when(i < n_i)
                def _():
                    pipe_body(i, j)
            return 0
        lax.fori_loop(0, (n_i + 1 + NSLOT - 1) // NSLOT, pair, 0)

        wait_scatter(0)
        wait_scatter(1)

        plsc.subcore_barrier()

        def flush(jj, _):
            r0 = base_row + jj * zrows
            pltpu.sync_copy(agg.at[pl.ds(r0, zrows)], stage)
            pltpu.sync_copy(stage, out_hbm.at[c, pl.ds(r0, zrows)])
            return 0
        lax.fori_loop(0, rows_per_tile // zrows, flush, 0)

    return body(pos_flat, feats_flat, senders, receivers)


def _tc_dense(agg2, bd, b32, n_nodes):
    rows = 2000
    grid = n_nodes // rows

    def body(a_ref, bd_ref, b_ref, o_ref):
        x = jnp.concatenate([a_ref[0], a_ref[1]], axis=-1)
        o_ref[...] = (jnp.dot(x, bd_ref[...],
                              preferred_element_type=jnp.float32)
                      + b_ref[...])

    return pl.pallas_call(
        body,
        grid=(grid,),
        in_specs=[
            pl.BlockSpec((NC, rows, 16), lambda i: (0, i, 0)),
            pl.BlockSpec((32, 32), lambda i: (0, 0)),
            pl.BlockSpec((1, 32), lambda i: (0, 0)),
        ],
        out_specs=pl.BlockSpec((rows, 32), lambda i: (i, 0)),
        out_shape=jax.ShapeDtypeStruct((n_nodes, 32), jnp.float32),
    )(agg2, bd, b32)


def kernel(positions, node_feats, senders, receivers, W, b):
    n_nodes = positions.shape[0]
    n_edges = senders.shape[0]
    n_f = node_feats.shape[1]
    pos4 = jnp.pad(positions, ((0, 0), (0, 1))).reshape(n_nodes // 4, 16)
    feats2 = node_feats.reshape(n_nodes * NC, n_f * 4 // NC)
    agg2 = _sc_aggregate(pos4, feats2, senders, receivers,
                         n_nodes, n_edges)
    bd = jnp.kron(jnp.eye(n_f, dtype=W.dtype), W) * (1.0 / 16.0)
    b32 = jnp.tile(b, n_f).reshape(1, n_f * 4)
    out32 = _tc_dense(agg2, bd, b32, n_nodes)
    return out32.reshape(n_nodes, n_f, 4)
